# Initial kernel scaffold; baseline (speedup 1.0000x reference)
import jax
import jax.numpy as jnp
from jax.experimental import pallas as pl

_B, _S, _DIM = 4, 2048, 6
_VOCAB, _D = 512, 1024
_ROWS = _B * (_S - 1)
_BLK = 512
_NBLK = 16


def _loss_kernel(idx_ref, emb_ref, w_ref, out_ref):
    i = pl.program_id(0)

    h = jnp.zeros((_BLK, _D), dtype=jnp.float32)
    for d in range(_DIM):
        ids = idx_ref[0, d].reshape(_BLK, 1)
        oh = (jax.lax.broadcasted_iota(jnp.int32, (_BLK, _VOCAB), 1)
              == ids).astype(jnp.bfloat16)
        h = h + jnp.dot(oh, emb_ref[d],
                        preferred_element_type=jnp.float32)

    t = jnp.tanh(h).astype(jnp.bfloat16)
    o = jnp.dot(t, w_ref[...], preferred_element_type=jnp.float32)
    diff = o - 1.0

    row = i * _BLK + jax.lax.broadcasted_iota(jnp.int32, (_BLK, _D), 0)
    diff = jnp.where(row < _ROWS, diff, 0.0)
    s = jnp.sum(diff * diff)

    @pl.when(i == 0)
    def _():
        out_ref[0, 0] = 0.0

    out_ref[0, 0] += s


def kernel(x, emb, W):
    xi = x[:, :-1].reshape(_ROWS, _DIM).astype(jnp.int32)
    idx = jnp.pad(xi, ((0, _NBLK * _BLK - _ROWS), (0, 0)))
    idx3 = idx.reshape(_NBLK, _BLK, _DIM).transpose(0, 2, 1)
    emb_bf = emb.astype(jnp.bfloat16)
    w_bf = W.astype(jnp.bfloat16)

    out = pl.pallas_call(
        _loss_kernel,
        grid=(_NBLK,),
        in_specs=[
            pl.BlockSpec((1, _DIM, _BLK), lambda i: (i, 0, 0)),
            pl.BlockSpec((_DIM, _VOCAB, _D), lambda i: (0, 0, 0)),
            pl.BlockSpec((_D, _D), lambda i: (0, 0)),
        ],
        out_specs=pl.BlockSpec((1, 1), lambda i: (0, 0)),
        out_shape=jax.ShapeDtypeStruct((1, 1), jnp.float32),
    )(idx3, emb_bf, w_bf)

    return out[0, 0] / (_ROWS * _D)


# speedup vs baseline: 5.1129x; 5.1129x over previous
import jax
import jax.numpy as jnp
from jax.experimental import pallas as pl

_B, _S, _DIM = 4, 2048, 6
_VOCAB, _D = 512, 1024
_ROWS = _B * (_S - 1)
_BLK = 512
_NBLK = 16


def _loss_kernel(idx_ref, emb_ref, w_ref, out_ref):
    i = pl.program_id(0)

    h = jnp.zeros((_BLK, _D), dtype=jnp.float32)
    for d in range(_DIM):
        ids = idx_ref[0, d].reshape(_BLK, 1)
        oh = (jax.lax.broadcasted_iota(jnp.int32, (_BLK, _VOCAB), 1)
              == ids).astype(jnp.bfloat16)
        h = h + jnp.dot(oh, emb_ref[d],
                        preferred_element_type=jnp.float32)

    t = jnp.tanh(h).astype(jnp.bfloat16)
    o = jnp.dot(t, w_ref[...], preferred_element_type=jnp.float32)
    diff = o - 1.0

    row = i * _BLK + jax.lax.broadcasted_iota(jnp.int32, (_BLK, _D), 0)
    diff = jnp.where(row < _ROWS, diff, 0.0)
    s = jnp.sum(diff * diff, keepdims=True)

    @pl.when(i == 0)
    def _():
        out_ref[...] = jnp.zeros((1, 1), jnp.float32)

    out_ref[...] += s


def kernel(x, emb, W):
    xi = x[:, :-1].reshape(_ROWS, _DIM).astype(jnp.int32)
    idx = jnp.pad(xi, ((0, _NBLK * _BLK - _ROWS), (0, 0)))
    idx3 = idx.reshape(_NBLK, _BLK, _DIM).transpose(0, 2, 1)
    emb_bf = emb.astype(jnp.bfloat16)
    w_bf = W.astype(jnp.bfloat16)

    out = pl.pallas_call(
        _loss_kernel,
        grid=(_NBLK,),
        in_specs=[
            pl.BlockSpec((1, _DIM, _BLK), lambda i: (i, 0, 0)),
            pl.BlockSpec((_DIM, _VOCAB, _D), lambda i: (0, 0, 0)),
            pl.BlockSpec((_D, _D), lambda i: (0, 0)),
        ],
        out_specs=pl.BlockSpec((1, 1), lambda i: (0, 0)),
        out_shape=jax.ShapeDtypeStruct((1, 1), jnp.float32),
    )(idx3, emb_bf, w_bf)

    return out[0, 0] / (_ROWS * _D)
